# Initial kernel scaffold; baseline (speedup 1.0000x reference)
import jax
import jax.numpy as jnp
from jax.experimental import pallas as pl


def kernel(text_token, image_token4, image_token8, image_token12, cache, W):
    raise NotImplementedError("write your pallas kernel here")


# speedup vs baseline: 2.3528x; 2.3528x over previous
import functools

import jax
import jax.numpy as jnp
from jax.experimental import pallas as pl

ALPHA = 0.2
MOMENTUM = 0.8
BQ = 512


def _rownorm(x):
    n = jnp.sqrt(jnp.sum(x * x, axis=1, keepdims=True))
    return x / jnp.clip(n, 1e-12)


def _img_stats(step, cache, tok_ref, rv_ref, as_ref, cm_ref, m):
    b = _rownorm(tok_ref[...])
    s = jax.lax.dot_general(b, cache, (((1,), (1,)), ((), ())))
    rv = jnp.max(s, axis=1, keepdims=True)
    jidx = jax.lax.broadcasted_iota(jnp.int32, s.shape, 1)
    amin = jnp.min(jnp.where(s == rv, jidx, m), axis=1, keepdims=True)
    rv_ref[...] = rv
    as_ref[...] = amin
    pcm = jnp.max(s, axis=0, keepdims=True)

    @pl.when(step == 0)
    def _():
        cm_ref[...] = pcm

    @pl.when(step != 0)
    def _():
        cm_ref[...] = jnp.maximum(cm_ref[...], pcm)


def _stats_kernel(text_ref, i4_ref, i8_ref, i12_ref, cache_ref, w1_ref, w2_ref,
                  tf_ref, loss_ref,
                  rv4_ref, as4_ref, cm4_ref,
                  rv8_ref, as8_ref, cm8_ref,
                  rv12_ref, as12_ref, cm12_ref, *, m):
    step = pl.program_id(0)
    cache = cache_ref[...]

    text = text_ref[...]
    base = _rownorm(text)
    s = jax.lax.dot_general(base, cache, (((1,), (1,)), ((), ())))
    p = jnp.exp(s - jnp.max(s, axis=1, keepdims=True))
    p = p / jnp.sum(p, axis=1, keepdims=True)
    fine = jax.lax.dot_general(p, cache, (((1,), (0,)), ((), ())))
    tf = ALPHA * (jax.lax.dot_general(text, w1_ref[...], (((1,), (1,)), ((), ())))
                  + jax.lax.dot_general(fine, w2_ref[...], (((1,), (1,)), ((), ())))) + text
    tf_ref[...] = tf

    ab = jnp.abs(_rownorm(tf) - text)
    pa = jnp.sum(jnp.sum(ab, axis=1, keepdims=True), axis=0, keepdims=True)

    @pl.when(step == 0)
    def _():
        loss_ref[...] = pa

    @pl.when(step != 0)
    def _():
        loss_ref[...] = loss_ref[...] + pa

    _img_stats(step, cache, i4_ref, rv4_ref, as4_ref, cm4_ref, m)
    _img_stats(step, cache, i8_ref, rv8_ref, as8_ref, cm8_ref, m)
    _img_stats(step, cache, i12_ref, rv12_ref, as12_ref, cm12_ref, m)


def _img_scatter(step, tok_ref, rv_ref, as_ref, cm_ref, sums_ref, cnt_ref, m, bq):
    b = _rownorm(tok_ref[...])
    a = as_ref[...]
    jidx = jax.lax.broadcasted_iota(jnp.int32, (bq, m), 1)
    mask = jidx == a
    cmg = jnp.max(jnp.where(mask, cm_ref[...], -jnp.inf), axis=1, keepdims=True)
    w = jnp.exp(rv_ref[...] - cmg)
    mf = mask.astype(jnp.float32)
    ps = jax.lax.dot_general(mf, w * b, (((0,), (0,)), ((), ())))
    pc = jnp.sum(mf, axis=0, keepdims=True)
    pcc = jax.lax.dot_general(mf, jnp.full((bq, 1), 1.0, jnp.float32),
                              (((0,), (0,)), ((), ())))
    del pc

    @pl.when(step == 0)
    def _():
        sums_ref[...] = ps
        cnt_ref[...] = pcc

    @pl.when(step != 0)
    def _():
        sums_ref[...] = sums_ref[...] + ps
        cnt_ref[...] = cnt_ref[...] + pcc


def _scatter_kernel(i4_ref, rv4_ref, as4_ref, cm4_ref,
                    i8_ref, rv8_ref, as8_ref, cm8_ref,
                    i12_ref, rv12_ref, as12_ref, cm12_ref,
                    s4_ref, c4_ref, s8_ref, c8_ref, s12_ref, c12_ref, *, m, bq):
    step = pl.program_id(0)
    _img_scatter(step, i4_ref, rv4_ref, as4_ref, cm4_ref, s4_ref, c4_ref, m, bq)
    _img_scatter(step, i8_ref, rv8_ref, as8_ref, cm8_ref, s8_ref, c8_ref, m, bq)
    _img_scatter(step, i12_ref, rv12_ref, as12_ref, cm12_ref, s12_ref, c12_ref, m, bq)


def _update_kernel(cache_ref, s4_ref, c4_ref, s8_ref, c8_ref, s12_ref, c12_ref,
                   out_ref):
    cache = cache_ref[...]
    acc = None
    for s_ref, c_ref in ((s4_ref, c4_ref), (s8_ref, c8_ref), (s12_ref, c12_ref)):
        upd = jnp.where(c_ref[...] > 0.0,
                        MOMENTUM * cache + (1.0 - MOMENTUM) * s_ref[...],
                        cache)
        u = _rownorm(upd)
        acc = u if acc is None else acc + u
    out_ref[...] = acc / 3.0


def kernel(text_token, image_token4, image_token8, image_token12, cache, W):
    c, d = text_token.shape
    m = cache.shape[0]
    nblk = c // BQ
    w1 = W[:, :d]
    w2 = W[:, d:]

    tok_spec = pl.BlockSpec((BQ, d), lambda i: (i, 0))
    full2 = lambda shape: pl.BlockSpec(shape, lambda i: (0, 0))
    col_spec = pl.BlockSpec((BQ, 1), lambda i: (i, 0))
    f32 = jnp.float32

    stats_out_shape = (
        jax.ShapeDtypeStruct((c, d), f32),
        jax.ShapeDtypeStruct((1, 1), f32),
    ) + tuple(
        x for _ in range(3) for x in (
            jax.ShapeDtypeStruct((c, 1), f32),
            jax.ShapeDtypeStruct((c, 1), jnp.int32),
            jax.ShapeDtypeStruct((1, m), f32),
        )
    )
    stats_out_spec = (
        tok_spec,
        full2((1, 1)),
    ) + tuple(
        x for _ in range(3) for x in (col_spec, col_spec, full2((1, m)))
    )

    (text_fine, loss_sum,
     rv4, as4, cm4, rv8, as8, cm8, rv12, as12, cm12) = pl.pallas_call(
        functools.partial(_stats_kernel, m=m),
        grid=(nblk,),
        in_specs=[tok_spec, tok_spec, tok_spec, tok_spec,
                  full2((m, d)), full2((d, d)), full2((d, d))],
        out_specs=stats_out_spec,
        out_shape=stats_out_shape,
    )(text_token, image_token4, image_token8, image_token12, cache, w1, w2)

    scat_out_shape = tuple(
        x for _ in range(3) for x in (
            jax.ShapeDtypeStruct((m, d), f32),
            jax.ShapeDtypeStruct((m, 1), f32),
        )
    )
    scat_out_spec = tuple(
        x for _ in range(3) for x in (full2((m, d)), full2((m, 1)))
    )

    (s4, c4, s8, c8, s12, c12) = pl.pallas_call(
        functools.partial(_scatter_kernel, m=m, bq=BQ),
        grid=(nblk,),
        in_specs=[tok_spec, col_spec, col_spec, full2((1, m)),
                  tok_spec, col_spec, col_spec, full2((1, m)),
                  tok_spec, col_spec, col_spec, full2((1, m))],
        out_specs=scat_out_spec,
        out_shape=scat_out_shape,
    )(image_token4, rv4, as4, cm4,
      image_token8, rv8, as8, cm8,
      image_token12, rv12, as12, cm12)

    bm = 688 if m % 688 == 0 else m
    row_spec = pl.BlockSpec((bm, d), lambda i: (i, 0))
    cnt_spec = pl.BlockSpec((bm, 1), lambda i: (i, 0))
    updated_cache = pl.pallas_call(
        _update_kernel,
        grid=(m // bm,),
        in_specs=[row_spec, row_spec, cnt_spec, row_spec, cnt_spec,
                  row_spec, cnt_spec],
        out_specs=row_spec,
        out_shape=jax.ShapeDtypeStruct((m, d), f32),
    )(cache, s4, c4, s8, c8, s12, c12)

    loss = loss_sum[0, 0] / (c * d)
    return (text_fine, loss, updated_cache)


# speedup vs baseline: 2.5894x; 1.1006x over previous
import functools

import jax
import jax.numpy as jnp
from jax.experimental import pallas as pl

ALPHA = 0.2
MOMENTUM = 0.8
BQ = 512


def _rownorm(x):
    n = jnp.sqrt(jnp.sum(x * x, axis=1, keepdims=True))
    return x / jnp.clip(n, 1e-12)


def _img_stats(step, cache, tok_ref, rv_ref, as_ref, cm_ref, cnt_ref, m):
    b = _rownorm(tok_ref[...])
    s = jax.lax.dot_general(b, cache, (((1,), (1,)), ((), ())))
    rv = jnp.max(s, axis=1, keepdims=True)
    jidx = jax.lax.broadcasted_iota(jnp.int32, s.shape, 1)
    amin = jnp.min(jnp.where(s == rv, jidx, m), axis=1, keepdims=True)
    rv_ref[...] = rv
    as_ref[...] = amin
    pcm = jnp.max(s, axis=0, keepdims=True)
    pc = jnp.sum((jidx == amin).astype(jnp.float32), axis=0, keepdims=True)

    @pl.when(step == 0)
    def _():
        cm_ref[...] = pcm
        cnt_ref[...] = pc

    @pl.when(step != 0)
    def _():
        cm_ref[...] = jnp.maximum(cm_ref[...], pcm)
        cnt_ref[...] = cnt_ref[...] + pc


def _stats_kernel(text_ref, i4_ref, i8_ref, i12_ref, cache_ref, w1_ref, w2_ref,
                  tf_ref, loss_ref,
                  rv4_ref, as4_ref, cm4_ref, cnt4_ref,
                  rv8_ref, as8_ref, cm8_ref, cnt8_ref,
                  rv12_ref, as12_ref, cm12_ref, cnt12_ref, *, m):
    step = pl.program_id(0)
    cache = cache_ref[...]

    text = text_ref[...]
    base = _rownorm(text)
    s = jax.lax.dot_general(base, cache, (((1,), (1,)), ((), ())))
    p = jnp.exp(s - jnp.max(s, axis=1, keepdims=True))
    p = p / jnp.sum(p, axis=1, keepdims=True)
    fine = jax.lax.dot_general(p, cache, (((1,), (0,)), ((), ())))
    tf = ALPHA * (jax.lax.dot_general(text, w1_ref[...], (((1,), (1,)), ((), ())))
                  + jax.lax.dot_general(fine, w2_ref[...], (((1,), (1,)), ((), ())))) + text
    tf_ref[...] = tf

    ab = jnp.abs(_rownorm(tf) - text)
    pa = jnp.sum(jnp.sum(ab, axis=1, keepdims=True), axis=0, keepdims=True)

    @pl.when(step == 0)
    def _():
        loss_ref[...] = pa

    @pl.when(step != 0)
    def _():
        loss_ref[...] = loss_ref[...] + pa

    _img_stats(step, cache, i4_ref, rv4_ref, as4_ref, cm4_ref, cnt4_ref, m)
    _img_stats(step, cache, i8_ref, rv8_ref, as8_ref, cm8_ref, cnt8_ref, m)
    _img_stats(step, cache, i12_ref, rv12_ref, as12_ref, cm12_ref, cnt12_ref, m)


def _img_scatter(step, tok_ref, rv_ref, as_ref, cm_ref, sums_ref, m, bq):
    b = _rownorm(tok_ref[...])
    a = as_ref[...]
    jidx = jax.lax.broadcasted_iota(jnp.int32, (bq, m), 1)
    mask = jidx == a
    cmg = jnp.max(jnp.where(mask, cm_ref[...], -jnp.inf), axis=1, keepdims=True)
    w = jnp.exp(rv_ref[...] - cmg)
    mf = mask.astype(jnp.bfloat16)
    wb = (w * b).astype(jnp.bfloat16)
    ps = jax.lax.dot_general(mf, wb, (((0,), (0,)), ((), ())),
                             preferred_element_type=jnp.float32)

    @pl.when(step == 0)
    def _():
        sums_ref[...] = ps

    @pl.when(step != 0)
    def _():
        sums_ref[...] = sums_ref[...] + ps


def _scatter_kernel(i4_ref, rv4_ref, as4_ref, cm4_ref,
                    i8_ref, rv8_ref, as8_ref, cm8_ref,
                    i12_ref, rv12_ref, as12_ref, cm12_ref,
                    s4_ref, s8_ref, s12_ref, *, m, bq):
    step = pl.program_id(0)
    _img_scatter(step, i4_ref, rv4_ref, as4_ref, cm4_ref, s4_ref, m, bq)
    _img_scatter(step, i8_ref, rv8_ref, as8_ref, cm8_ref, s8_ref, m, bq)
    _img_scatter(step, i12_ref, rv12_ref, as12_ref, cm12_ref, s12_ref, m, bq)


def _update_kernel(cache_ref, s4_ref, c4_ref, s8_ref, c8_ref, s12_ref, c12_ref,
                   out_ref):
    cache = cache_ref[...]
    acc = None
    for s_ref, c_ref in ((s4_ref, c4_ref), (s8_ref, c8_ref), (s12_ref, c12_ref)):
        upd = jnp.where(c_ref[...] > 0.0,
                        MOMENTUM * cache + (1.0 - MOMENTUM) * s_ref[...],
                        cache)
        u = _rownorm(upd)
        acc = u if acc is None else acc + u
    out_ref[...] = acc / 3.0


def kernel(text_token, image_token4, image_token8, image_token12, cache, W):
    c, d = text_token.shape
    m = cache.shape[0]
    nblk = c // BQ
    w1 = W[:, :d]
    w2 = W[:, d:]

    tok_spec = pl.BlockSpec((BQ, d), lambda i: (i, 0))
    full2 = lambda shape: pl.BlockSpec(shape, lambda i: (0, 0))
    col_spec = pl.BlockSpec((BQ, 1), lambda i: (i, 0))
    f32 = jnp.float32

    stats_out_shape = (
        jax.ShapeDtypeStruct((c, d), f32),
        jax.ShapeDtypeStruct((1, 1), f32),
    ) + tuple(
        x for _ in range(3) for x in (
            jax.ShapeDtypeStruct((c, 1), f32),
            jax.ShapeDtypeStruct((c, 1), jnp.int32),
            jax.ShapeDtypeStruct((1, m), f32),
            jax.ShapeDtypeStruct((1, m), f32),
        )
    )
    stats_out_spec = (
        tok_spec,
        full2((1, 1)),
    ) + tuple(
        x for _ in range(3) for x in (col_spec, col_spec, full2((1, m)),
                                      full2((1, m)))
    )

    (text_fine, loss_sum,
     rv4, as4, cm4, cnt4, rv8, as8, cm8, cnt8,
     rv12, as12, cm12, cnt12) = pl.pallas_call(
        functools.partial(_stats_kernel, m=m),
        grid=(nblk,),
        in_specs=[tok_spec, tok_spec, tok_spec, tok_spec,
                  full2((m, d)), full2((d, d)), full2((d, d))],
        out_specs=stats_out_spec,
        out_shape=stats_out_shape,
    )(text_token, image_token4, image_token8, image_token12, cache, w1, w2)

    scat_out_shape = tuple(jax.ShapeDtypeStruct((m, d), f32) for _ in range(3))
    scat_out_spec = tuple(full2((m, d)) for _ in range(3))

    (s4, s8, s12) = pl.pallas_call(
        functools.partial(_scatter_kernel, m=m, bq=BQ),
        grid=(nblk,),
        in_specs=[tok_spec, col_spec, col_spec, full2((1, m)),
                  tok_spec, col_spec, col_spec, full2((1, m)),
                  tok_spec, col_spec, col_spec, full2((1, m))],
        out_specs=scat_out_spec,
        out_shape=scat_out_shape,
    )(image_token4, rv4, as4, cm4,
      image_token8, rv8, as8, cm8,
      image_token12, rv12, as12, cm12)

    bm = 688 if m % 688 == 0 else m
    row_spec = pl.BlockSpec((bm, d), lambda i: (i, 0))
    cnt_spec = pl.BlockSpec((bm, 1), lambda i: (i, 0))
    c4t, c8t, c12t = cnt4.T, cnt8.T, cnt12.T
    updated_cache = pl.pallas_call(
        _update_kernel,
        grid=(m // bm,),
        in_specs=[row_spec, row_spec, cnt_spec, row_spec, cnt_spec,
                  row_spec, cnt_spec],
        out_specs=row_spec,
        out_shape=jax.ShapeDtypeStruct((m, d), f32),
    )(cache, s4, c4t, s8, c8t, s12, c12t)

    loss = loss_sum[0, 0] / (c * d)
    return (text_fine, loss, updated_cache)
